# baseline (speedup 1.0000x reference)
import jax
import jax.numpy as jnp
from jax import lax
from jax.experimental import pallas as pl
from jax.experimental.pallas import tpu as pltpu
from jax.experimental.pallas import tpu_sc as plsc

B, S, D = 4, 4096, 2048
NC, NS = 2, 16
NW = NC * NS
ROWS_W = S // NW
R = 4
NSTEP = ROWS_W // R
U = 4
NVI = D // (16 * U)
NSLOT = 3


def _sc_body(x_hbm, t_hbm, o_hbm, *scratch):
    tb = scratch[0:NSLOT]
    xb = scratch[NSLOT:2 * NSLOT]
    tsem = scratch[2 * NSLOT:3 * NSLOT]
    xsem = scratch[3 * NSLOT:4 * NSLOT]

    wid = lax.axis_index("s") * NC + lax.axis_index("c")
    base_row = wid * ROWS_W

    def issue_loads(s):
        slot = s % NSLOT
        row0 = base_row + s * R
        th = pltpu.async_copy(t_hbm.at[pl.ds(row0, R), :], tb[slot],
                              tsem[slot])
        xh = pltpu.async_copy(x_hbm.at[:, pl.ds(row0, R), :], xb[slot],
                              xsem[slot])
        return th, xh

    loads = {0: issue_loads(0), 1: issue_loads(1)}
    stores = {}
    for s in range(NSTEP):
        slot = s % NSLOT
        if s + 2 < NSTEP:
            if (s - 1) in stores:
                stores.pop(s - 1).wait()
            loads[s + 2] = issue_loads(s + 2)
        th, xh = loads.pop(s)
        th.wait()
        xh.wait()

        tbs = tb[slot]
        xbs = xb[slot]

        def vec_body(i, _, tbs=tbs, xbs=xbs):
            r = i // NVI
            j = i % NVI
            for u in range(U):
                o = j * (U * 16) + u * 16
                tv = tbs[r, pl.ds(o, 16)]
                for b in range(B):
                    xbs[b, r, pl.ds(o, 16)] = xbs[b, r, pl.ds(o, 16)] + tv
            return 0

        lax.fori_loop(0, R * NVI, vec_body, 0)

        row0 = base_row + s * R
        stores[s] = pltpu.async_copy(xb[slot],
                                     o_hbm.at[:, pl.ds(row0, R), :],
                                     xsem[slot])
    for s in range(NSTEP - 3, NSTEP):
        if s in stores:
            stores.pop(s).wait()


def kernel(inputs, position_table):
    scratch = (
        [pltpu.VMEM((R, D), jnp.float32) for _ in range(NSLOT)]
        + [pltpu.VMEM((B, R, D), jnp.float32) for _ in range(NSLOT)]
        + [pltpu.SemaphoreType.DMA for _ in range(2 * NSLOT)]
    )
    k = pl.kernel(
        _sc_body,
        out_type=jax.ShapeDtypeStruct((B, S, D), jnp.float32),
        mesh=plsc.VectorSubcoreMesh(core_axis_name="c", subcore_axis_name="s"),
        scratch_types=scratch,
    )
    return k(inputs, position_table)


# speedup vs baseline: 1.3184x; 1.3184x over previous
import jax
import jax.numpy as jnp
from jax import lax
from jax.experimental import pallas as pl
from jax.experimental.pallas import tpu as pltpu
from jax.experimental.pallas import tpu_sc as plsc

B, S, D = 4, 4096, 2048
NC, NS = 2, 16
NW = NC * NS
ROWS_W = S // NW
R = 2
NSTEP = ROWS_W // R
U = 4
NVI = D // (16 * U)
NSLOT = 4


def _sc_body(x_hbm, t_hbm, o_hbm, *scratch):
    tb = scratch[0:NSLOT]
    xb = [scratch[NSLOT + B * k: NSLOT + B * k + B] for k in range(NSLOT)]
    nbuf = NSLOT * (1 + B)
    tsem = scratch[nbuf: nbuf + NSLOT]
    xsem = [scratch[nbuf + NSLOT + B * k: nbuf + NSLOT + B * k + B]
            for k in range(NSLOT)]

    wid = lax.axis_index("s") * NC + lax.axis_index("c")
    base_row = wid * ROWS_W

    def issue_loads(s):
        slot = s % NSLOT
        row0 = base_row + s * R
        th = pltpu.async_copy(t_hbm.at[pl.ds(row0, R), :], tb[slot],
                              tsem[slot])
        xhs = [
            pltpu.async_copy(x_hbm.at[b, pl.ds(row0, R), :],
                             xb[slot][b], xsem[slot][b])
            for b in range(B)
        ]
        return th, xhs

    loads = {0: issue_loads(0), 1: issue_loads(1)}
    stores = {}
    for s in range(NSTEP):
        slot = s % NSLOT
        if s + 2 < NSTEP:
            if (s - 2) in stores:
                for h in stores.pop(s - 2):
                    h.wait()
            loads[s + 2] = issue_loads(s + 2)
        th, xhs = loads.pop(s)
        th.wait()
        for h in xhs:
            h.wait()

        tbs = tb[slot]
        xbs = xb[slot]

        def vec_body(i, _, tbs=tbs, xbs=xbs):
            r = i // NVI
            j = i % NVI
            for u in range(U):
                o = j * (U * 16) + u * 16
                tv = tbs[r, pl.ds(o, 16)]
                for b in range(B):
                    xbs[b][r, pl.ds(o, 16)] = xbs[b][r, pl.ds(o, 16)] + tv
            return 0

        lax.fori_loop(0, R * NVI, vec_body, 0)

        row0 = base_row + s * R
        stores[s] = [
            pltpu.async_copy(xb[slot][b],
                             o_hbm.at[b, pl.ds(row0, R), :],
                             xsem[slot][b])
            for b in range(B)
        ]
    for s in sorted(stores):
        for h in stores[s]:
            h.wait()


def kernel(inputs, position_table):
    scratch = (
        [pltpu.VMEM((R, D), jnp.float32) for _ in range(NSLOT * (1 + B))]
        + [pltpu.SemaphoreType.DMA for _ in range(NSLOT * (1 + B))]
    )
    k = pl.kernel(
        _sc_body,
        out_type=jax.ShapeDtypeStruct((B, S, D), jnp.float32),
        mesh=plsc.VectorSubcoreMesh(core_axis_name="c", subcore_axis_name="s"),
        scratch_types=scratch,
    )
    return k(inputs, position_table)


# speedup vs baseline: 1.3686x; 1.0381x over previous
import jax
import jax.numpy as jnp
from jax import lax
from jax.experimental import pallas as pl
from jax.experimental.pallas import tpu as pltpu
from jax.experimental.pallas import tpu_sc as plsc

B, S, D = 4, 4096, 2048
NC, NS = 2, 16
NW = NC * NS
ROWS_W = S // NW
R = 4
NSTEP = ROWS_W // R
U = 4
NVI = D // (16 * U)
NSLOT = 3


def _sc_body(x_hbm, t_hbm, o_hbm, *scratch):
    tb = scratch[0:NSLOT]
    xb = [scratch[NSLOT + B * k: NSLOT + B * k + B] for k in range(NSLOT)]
    nbuf = NSLOT * (1 + B)
    tsem = scratch[nbuf: nbuf + NSLOT]
    xsem = [scratch[nbuf + NSLOT + B * k: nbuf + NSLOT + B * k + B]
            for k in range(NSLOT)]

    wid = lax.axis_index("s") * NC + lax.axis_index("c")
    base_row = wid * ROWS_W

    def issue_loads(s):
        slot = s % NSLOT
        row0 = base_row + s * R
        th = pltpu.async_copy(t_hbm.at[pl.ds(row0, R), :], tb[slot],
                              tsem[slot])
        xhs = [
            pltpu.async_copy(x_hbm.at[b, pl.ds(row0, R), :],
                             xb[slot][b], xsem[slot][b])
            for b in range(B)
        ]
        return th, xhs

    loads = {0: issue_loads(0), 1: issue_loads(1)}
    stores = {}
    for s in range(NSTEP):
        slot = s % NSLOT
        if s + 2 < NSTEP:
            if (s - 1) in stores:
                for h in stores[s - 1]:
                    h.wait()
            loads[s + 2] = issue_loads(s + 2)
        th, xhs = loads.pop(s)
        th.wait()
        for h in xhs:
            h.wait()

        tbs = tb[slot]
        xbs = xb[slot]

        def vec_body(i, _, tbs=tbs, xbs=xbs):
            r = i // NVI
            j = i % NVI
            for u in range(U):
                o = j * (U * 16) + u * 16
                tv = tbs[r, pl.ds(o, 16)]
                for b in range(B):
                    xbs[b][r, pl.ds(o, 16)] = xbs[b][r, pl.ds(o, 16)] + tv
            return 0

        lax.fori_loop(0, R * NVI, vec_body, 0)

        row0 = base_row + s * R
        stores[s] = [
            pltpu.async_copy(xb[slot][b],
                             o_hbm.at[b, pl.ds(row0, R), :],
                             xsem[slot][b])
            for b in range(B)
        ]
    for s in range(NSTEP - 3, NSTEP):
        for h in stores[s]:
            h.wait()


def kernel(inputs, position_table):
    scratch = (
        [pltpu.VMEM((R, D), jnp.float32) for _ in range(NSLOT * (1 + B))]
        + [pltpu.SemaphoreType.DMA for _ in range(NSLOT * (1 + B))]
    )
    k = pl.kernel(
        _sc_body,
        out_type=jax.ShapeDtypeStruct((B, S, D), jnp.float32),
        mesh=plsc.VectorSubcoreMesh(core_axis_name="c", subcore_axis_name="s"),
        scratch_types=scratch,
    )
    return k(inputs, position_table)
